# baseline (speedup 1.0000x reference)
import jax
import jax.numpy as jnp
from jax import lax
from jax.experimental import pallas as pl
from jax.experimental.pallas import tpu as pltpu
from jax.experimental.pallas import tpu_sc as plsc

N = 10000
E = 320000
F_IN = 128
H = 128
C_OUT = 47
D2 = 48

NC = 2
NS = 16
NW = NC * NS
LANES = 16

N_PAD = 10240
ROWS_PER_TILE = N_PAD // NS

CHUNK = 128
NCHUNKS = E // CHUNK
BASE_CH = NCHUNKS // NW
EXTRA_CH = NCHUNKS - BASE_CH * NW
MAX_CH = BASE_CH + 1


def _zero_fill(buf, rows, d):
    zero = jnp.zeros((LANES,), jnp.float32)

    @pl.loop(0, rows)
    def _(i):
        @pl.loop(0, d // LANES)
        def _(j):
            buf[i, pl.ds(j * LANES, LANES)] = zero


def _zero_stripe(buf, rows, acc, r0):
    @pl.loop(0, ROWS_PER_TILE // rows)
    def _(k):
        pltpu.sync_copy(buf.at[pl.ds(0, rows)],
                        acc.at[pl.ds(r0 + k * rows, rows)])


def _nch_of(wid):
    return BASE_CH + jnp.where(wid < EXTRA_CH, 1, 0)


def _pack_ei(edge_index, chunk):
    nck = E // chunk
    e3 = edge_index.reshape(2, nck, chunk).transpose(1, 0, 2)
    return jnp.pad(e3.reshape(2 * nck, chunk), ((0, 8), (0, 0)))


UNROLL = 16
NBAT = 4


def _make_sc_agg(d, nbuf, chunk=128):
    assert UNROLL % nbuf == 0 and d % LANES == 0
    nchunks = E // chunk
    base_ch = nchunks // NW
    extra_ch = nchunks - base_ch * NW
    max_ch = base_ch + 1

    mesh = plsc.VectorSubcoreMesh(core_axis_name="c", subcore_axis_name="s")
    out_type = [jax.ShapeDtypeStruct((NC, N_PAD, d), jnp.float32)]
    scratch = (
        [pltpu.VMEM_SHARED((N_PAD, d), jnp.float32)]
        + [pltpu.VMEM((chunk, d), jnp.float32) for _ in range(nbuf)]
        + [pltpu.VMEM((8, chunk), jnp.int32) for _ in range(NBAT)]
        + [pltpu.SemaphoreType.DMA for _ in range(2 * nbuf + NBAT)]
    )

    def body(xw, ei, agg_out, acc, *refs):
        rows = refs[:nbuf]
        idxb = refs[nbuf:nbuf + NBAT]
        sem_g = refs[nbuf + NBAT:2 * nbuf + NBAT]
        sem_s = refs[2 * nbuf + NBAT:3 * nbuf + NBAT]
        sem_i = refs[3 * nbuf + NBAT:]
        c = lax.axis_index("c")
        s = lax.axis_index("s")
        wid = c * NS + s
        nch = (base_ch + jnp.where(wid < extra_ch, 1, 0)
               if extra_ch else base_ch)
        start_w = wid * base_ch + jnp.minimum(wid, extra_ch)

        _zero_fill(rows[0], 128, d)
        r0 = s * ROWS_PER_TILE
        _zero_stripe(rows[0], 128, acc, r0)
        plsc.subcore_barrier()

        def issue_batch(k_first, ring):
            base = 2 * (start_w + k_first)
            pltpu.async_copy(ei.at[pl.ds(base, 8)], idxb[ring], sem_i[ring])

        def wait_batch(k_first, ring):
            base = 2 * (start_w + k_first)
            pltpu.make_async_copy(ei.at[pl.ds(base, 8)], idxb[ring],
                                  sem_i[ring]).wait()

        def _sidx(ring, t):
            return idxb[ring].at[2 * t]

        def _didx(ring, t):
            return idxb[ring].at[2 * t + 1]

        def issue_gather(b, ring, t):
            pltpu.async_copy(xw.at[_sidx(ring, t)], rows[b], sem_g[b])

        def wait_gather(b, ring, t):
            pltpu.make_async_copy(xw.at[_sidx(ring, t)], rows[b],
                                  sem_g[b]).wait()

        def issue_scatter(b, ring, t):
            pltpu.async_copy(rows[b], acc.at[_didx(ring, t)],
                             sem_s[b], add=True)

        def wait_scatter(b, ring, t):
            pltpu.make_async_copy(rows[b], acc.at[_didx(ring, t)],
                                  sem_s[b]).wait()

        issue_batch(0, 0)
        issue_batch(4, 1)
        wait_batch(0, 0)
        issue_gather(0, 0, 0)

        nouter = (max_ch + nbuf + UNROLL - 1) // UNROLL

        @pl.loop(0, nouter)
        def _(io):
            k0 = io * UNROLL
            for j in range(UNROLL):
                k = k0 + j
                b = j % nbuf
                ring = (j // 4) % NBAT
                t = j % 4
                jn = (j + 1) % UNROLL
                jp = (j + 1 - nbuf) % UNROLL

                @pl.when(k < nch)
                def _():
                    wait_gather(b, ring, t)
                    issue_scatter(b, ring, t)

                @pl.when((k + 1 - nbuf >= 0) & (k + 1 - nbuf < nch))
                def _():
                    wait_scatter((j + 1) % nbuf, (jp // 4) % NBAT, jp % 4)

                if (j + 1) % 4 == 0:
                    @pl.when(k + 1 < nch)
                    def _():
                        wait_batch(k + 1, (jn // 4) % NBAT)

                    @pl.when(k + 5 < nch)
                    def _():
                        issue_batch(k + 5, ((j + 1) // 4 + 1) % NBAT)

                @pl.when(k + 1 < nch)
                def _():
                    issue_gather((j + 1) % nbuf, (jn // 4) % NBAT, jn % 4)

        plsc.subcore_barrier()

        pltpu.sync_copy(acc.at[pl.ds(r0, ROWS_PER_TILE)],
                        agg_out.at[c, pl.ds(r0, ROWS_PER_TILE)])

    return pl.kernel(
        body, out_type=out_type, mesh=mesh, scratch_types=scratch,
        compiler_params=pltpu.CompilerParams(use_tc_tiling_on_sc=False))


def _make_sc_deg(chunk=512, nsem=4, nidx=8):
    nchunks = E // chunk
    base_ch = nchunks // NW
    extra_ch = nchunks - base_ch * NW
    max_ch = base_ch + 1
    mesh = plsc.VectorSubcoreMesh(core_axis_name="c", subcore_axis_name="s")
    out_type = [jax.ShapeDtypeStruct((NC, N_PAD, LANES), jnp.float32)]
    scratch = (
        [pltpu.VMEM_SHARED((N_PAD, LANES), jnp.float32)]
        + [pltpu.VMEM((chunk, LANES), jnp.float32)]
        + [pltpu.VMEM((chunk,), jnp.int32) for _ in range(nidx)]
        + [pltpu.SemaphoreType.DMA for _ in range(nsem + nidx)]
    )

    def body(ei, deg_out, dacc, ones, *refs):
        idx = refs[:nidx]
        sem_s = refs[nidx:nidx + nsem]
        sem_i = refs[nidx + nsem:]
        c = lax.axis_index("c")
        s = lax.axis_index("s")
        wid = c * NS + s
        nch = (base_ch + jnp.where(wid < extra_ch, 1, 0)
               if extra_ch else base_ch)
        start_w = wid * base_ch + jnp.minimum(wid, extra_ch)

        _zero_fill(ones, 128, LANES)
        r0 = s * ROWS_PER_TILE
        _zero_stripe(ones, 128, dacc, r0)

        one = jnp.zeros((LANES,), jnp.float32) + 1.0

        @pl.loop(0, chunk)
        def _(i):
            ones[i, pl.ds(0, LANES)] = one

        plsc.subcore_barrier()

        def issue_idx(k, ib):
            pltpu.async_copy(ei.at[2 * (start_w + k) + 1], idx[ib],
                             sem_i[ib])

        def wait_idx(k, ib):
            pltpu.make_async_copy(ei.at[2 * (start_w + k) + 1], idx[ib],
                                  sem_i[ib]).wait()

        def issue_scatter(sb, ib):
            pltpu.async_copy(ones, dacc.at[idx[ib]], sem_s[sb], add=True)

        def wait_scatter(sb, ib):
            pltpu.make_async_copy(ones, dacc.at[idx[ib]], sem_s[sb]).wait()

        issue_idx(0, 0)
        issue_idx(1, 1)

        nouter = (max_ch + nsem + nidx - 1) // nidx

        @pl.loop(0, nouter)
        def _(io):
            k0 = io * nidx
            for j in range(nidx):
                k = k0 + j
                sb = j % nsem

                @pl.when((k >= nsem) & (k - nsem < nch))
                def _():
                    wait_scatter(sb, (j - nsem) % nidx)

                @pl.when(k < nch)
                def _():
                    wait_idx(k, j)
                    issue_scatter(sb, j)

                @pl.when(k + 2 < nch)
                def _():
                    issue_idx(k + 2, (j + 2) % nidx)

        plsc.subcore_barrier()
        pltpu.sync_copy(dacc.at[pl.ds(r0, ROWS_PER_TILE)],
                        deg_out.at[c, pl.ds(r0, ROWS_PER_TILE)])

    return pl.kernel(
        body, out_type=out_type, mesh=mesh, scratch_types=scratch,
        compiler_params=pltpu.CompilerParams(use_tc_tiling_on_sc=False))


_sc_agg1 = _make_sc_agg(H, 2, 160)
_sc_agg2 = _make_sc_agg(D2, 2, 512)
_sc_deg = _make_sc_deg()

TC_BLK = 2000


def _lin2_body(x_ref, wl_ref, wr_ref, b_ref, xl_ref, xr_ref):
    x = x_ref[...]
    xl_ref[...] = jnp.dot(x, wl_ref[...], preferred_element_type=jnp.float32)
    xr_ref[...] = (jnp.dot(x, wr_ref[...], preferred_element_type=jnp.float32)
                   + b_ref[...])


def _lin2(x, wl, wr, b):
    return pl.pallas_call(
        _lin2_body,
        grid=(N // TC_BLK,),
        in_specs=[
            pl.BlockSpec((TC_BLK, F_IN), lambda i: (i, 0)),
            pl.BlockSpec((F_IN, H), lambda i: (0, 0)),
            pl.BlockSpec((F_IN, H), lambda i: (0, 0)),
            pl.BlockSpec((1, H), lambda i: (0, 0)),
        ],
        out_specs=[
            pl.BlockSpec((TC_BLK, H), lambda i: (i, 0)),
            pl.BlockSpec((TC_BLK, H), lambda i: (i, 0)),
        ],
        out_shape=[
            jax.ShapeDtypeStruct((N, H), jnp.float32),
            jax.ShapeDtypeStruct((N, H), jnp.float32),
        ],
    )(x, wl, wr, b)


def _mid_body(aggp_ref, degp_ref, xr_ref, w2l_ref, w2r_ref, b2_ref,
              hl_ref, hr_ref):
    agg = aggp_ref[0] + aggp_ref[1]
    deg = degp_ref[0, :, 0:1] + degp_ref[1, :, 0:1]
    h = jnp.maximum(agg / jnp.maximum(deg, 1.0) + xr_ref[...], 0.0)
    zcol = jnp.zeros((H, 1), jnp.float32)
    w2l = jnp.concatenate([w2l_ref[...], zcol], axis=1)
    w2r = jnp.concatenate([w2r_ref[...], zcol], axis=1)
    b2 = jnp.concatenate([b2_ref[...], jnp.zeros((1, 1), jnp.float32)], axis=1)
    hl_ref[...] = jnp.dot(h, w2l, preferred_element_type=jnp.float32)
    hr_ref[...] = jnp.dot(h, w2r, preferred_element_type=jnp.float32) + b2


def _mid(aggp, degp, xr, w2l, w2r, b2):
    return pl.pallas_call(
        _mid_body,
        grid=(N // TC_BLK,),
        in_specs=[
            pl.BlockSpec((NC, TC_BLK, H), lambda i: (0, i, 0)),
            pl.BlockSpec((NC, TC_BLK, LANES), lambda i: (0, i, 0)),
            pl.BlockSpec((TC_BLK, H), lambda i: (i, 0)),
            pl.BlockSpec((H, C_OUT), lambda i: (0, 0)),
            pl.BlockSpec((H, C_OUT), lambda i: (0, 0)),
            pl.BlockSpec((1, C_OUT), lambda i: (0, 0)),
        ],
        out_specs=[
            pl.BlockSpec((TC_BLK, D2), lambda i: (i, 0)),
            pl.BlockSpec((TC_BLK, D2), lambda i: (i, 0)),
        ],
        out_shape=[
            jax.ShapeDtypeStruct((N, D2), jnp.float32),
            jax.ShapeDtypeStruct((N, D2), jnp.float32),
        ],
    )(aggp, degp, xr, w2l, w2r, b2)


def _out_body(aggp_ref, degp_ref, hr_ref, o_ref):
    agg = aggp_ref[0] + aggp_ref[1]
    deg = degp_ref[0, :, 0:1] + degp_ref[1, :, 0:1]
    res = agg / jnp.maximum(deg, 1.0) + hr_ref[...]
    o_ref[...] = res[:, :C_OUT]


def _outk(agg2p, degp, hr):
    return pl.pallas_call(
        _out_body,
        grid=(N // TC_BLK,),
        in_specs=[
            pl.BlockSpec((NC, TC_BLK, D2), lambda i: (0, i, 0)),
            pl.BlockSpec((NC, TC_BLK, LANES), lambda i: (0, i, 0)),
            pl.BlockSpec((TC_BLK, D2), lambda i: (i, 0)),
        ],
        out_specs=pl.BlockSpec((TC_BLK, C_OUT), lambda i: (i, 0)),
        out_shape=jax.ShapeDtypeStruct((N, C_OUT), jnp.float32),
    )(agg2p, degp, hr)


@jax.jit
def kernel(x, edge_index, W1l, W1r, b1, W2l, W2r, b2):
    ei160 = _pack_ei(edge_index, 160)
    ei512 = _pack_ei(edge_index, 512)
    degp, = _sc_deg(ei512)
    xl, xr = _lin2(x, W1l, W1r, b1.reshape(1, H))
    agg1p, = _sc_agg1(xl, ei160)
    hl, hr = _mid(agg1p, degp, xr, W2l, W2r, b2.reshape(1, C_OUT))
    agg2p, = _sc_agg2(hl, ei512)
    return _outk(agg2p, degp, hr)


# speedup vs baseline: 1.0319x; 1.0319x over previous
import jax
import jax.numpy as jnp
from jax import lax
from jax.experimental import pallas as pl
from jax.experimental.pallas import tpu as pltpu
from jax.experimental.pallas import tpu_sc as plsc

N = 10000
E = 320000
F_IN = 128
H = 128
C_OUT = 47
D2 = 48

NC = 2
NS = 16
NW = NC * NS
LANES = 16

N_PAD = 10240
ROWS_PER_TILE = N_PAD // NS

CHUNK = 128
NCHUNKS = E // CHUNK
BASE_CH = NCHUNKS // NW
EXTRA_CH = NCHUNKS - BASE_CH * NW
MAX_CH = BASE_CH + 1


def _zero_fill(buf, rows, d):
    zero = jnp.zeros((LANES,), jnp.float32)

    @pl.loop(0, rows)
    def _(i):
        @pl.loop(0, d // LANES)
        def _(j):
            buf[i, pl.ds(j * LANES, LANES)] = zero


def _zero_stripe(buf, rows, acc, r0):
    @pl.loop(0, ROWS_PER_TILE // rows)
    def _(k):
        pltpu.sync_copy(buf.at[pl.ds(0, rows)],
                        acc.at[pl.ds(r0 + k * rows, rows)])


def _nch_of(wid):
    return BASE_CH + jnp.where(wid < EXTRA_CH, 1, 0)


def _pack_ei(edge_index, chunk):
    nck = E // chunk
    e3 = edge_index.reshape(2, nck, chunk).transpose(1, 0, 2)
    return jnp.pad(e3.reshape(2 * nck, chunk), ((0, 8), (0, 0)))


UNROLL = 16
NBAT = 4


def _make_sc_agg(d, nbuf, chunk=128):
    assert UNROLL % nbuf == 0 and d % LANES == 0
    nchunks = E // chunk
    base_ch = nchunks // NW
    extra_ch = nchunks - base_ch * NW
    max_ch = base_ch + 1

    mesh = plsc.VectorSubcoreMesh(core_axis_name="c", subcore_axis_name="s")
    out_type = [jax.ShapeDtypeStruct((NC, N_PAD, d), jnp.float32)]
    scratch = (
        [pltpu.VMEM_SHARED((N_PAD, d), jnp.float32)]
        + [pltpu.VMEM((chunk, d), jnp.float32) for _ in range(nbuf)]
        + [pltpu.VMEM((8, chunk), jnp.int32) for _ in range(NBAT)]
        + [pltpu.SemaphoreType.DMA for _ in range(2 * nbuf + NBAT)]
    )

    def body(xw, ei, agg_out, acc, *refs):
        rows = refs[:nbuf]
        idxb = refs[nbuf:nbuf + NBAT]
        sem_g = refs[nbuf + NBAT:2 * nbuf + NBAT]
        sem_s = refs[2 * nbuf + NBAT:3 * nbuf + NBAT]
        sem_i = refs[3 * nbuf + NBAT:]
        c = lax.axis_index("c")
        s = lax.axis_index("s")
        wid = c * NS + s
        nch = (base_ch + jnp.where(wid < extra_ch, 1, 0)
               if extra_ch else base_ch)
        start_w = wid * base_ch + jnp.minimum(wid, extra_ch)

        _zero_fill(rows[0], 128, d)
        r0 = s * ROWS_PER_TILE
        _zero_stripe(rows[0], 128, acc, r0)
        plsc.subcore_barrier()

        def issue_batch(k_first, ring):
            base = 2 * (start_w + k_first)
            pltpu.async_copy(ei.at[pl.ds(base, 8)], idxb[ring], sem_i[ring])

        def wait_batch(k_first, ring):
            base = 2 * (start_w + k_first)
            pltpu.make_async_copy(ei.at[pl.ds(base, 8)], idxb[ring],
                                  sem_i[ring]).wait()

        def _sidx(ring, t):
            return idxb[ring].at[2 * t]

        def _didx(ring, t):
            return idxb[ring].at[2 * t + 1]

        def issue_gather(b, ring, t):
            pltpu.async_copy(xw.at[_sidx(ring, t)], rows[b], sem_g[b])

        def wait_gather(b, ring, t):
            pltpu.make_async_copy(xw.at[_sidx(ring, t)], rows[b],
                                  sem_g[b]).wait()

        def issue_scatter(b, ring, t):
            pltpu.async_copy(rows[b], acc.at[_didx(ring, t)],
                             sem_s[b], add=True)

        def wait_scatter(b, ring, t):
            pltpu.make_async_copy(rows[b], acc.at[_didx(ring, t)],
                                  sem_s[b]).wait()

        issue_batch(0, 0)
        issue_batch(4, 1)
        wait_batch(0, 0)
        issue_gather(0, 0, 0)

        nouter = (max_ch + nbuf + UNROLL - 1) // UNROLL

        @pl.loop(0, nouter)
        def _(io):
            k0 = io * UNROLL
            for j in range(UNROLL):
                k = k0 + j
                b = j % nbuf
                ring = (j // 4) % NBAT
                t = j % 4
                jn = (j + 1) % UNROLL
                jp = (j + 1 - nbuf) % UNROLL

                @pl.when(k < nch)
                def _():
                    wait_gather(b, ring, t)
                    issue_scatter(b, ring, t)

                @pl.when((k + 1 - nbuf >= 0) & (k + 1 - nbuf < nch))
                def _():
                    wait_scatter((j + 1) % nbuf, (jp // 4) % NBAT, jp % 4)

                if (j + 1) % 4 == 0:
                    @pl.when(k + 1 < nch)
                    def _():
                        wait_batch(k + 1, (jn // 4) % NBAT)

                    @pl.when(k + 5 < nch)
                    def _():
                        issue_batch(k + 5, ((j + 1) // 4 + 1) % NBAT)

                @pl.when(k + 1 < nch)
                def _():
                    issue_gather((j + 1) % nbuf, (jn // 4) % NBAT, jn % 4)

        plsc.subcore_barrier()

        pltpu.sync_copy(acc.at[pl.ds(r0, ROWS_PER_TILE)],
                        agg_out.at[c, pl.ds(r0, ROWS_PER_TILE)])

    return pl.kernel(
        body, out_type=out_type, mesh=mesh, scratch_types=scratch,
        compiler_params=pltpu.CompilerParams(use_tc_tiling_on_sc=False))


def _make_sc_deg(chunk=512, nsem=4, nidx=8):
    nchunks = E // chunk
    base_ch = nchunks // NW
    extra_ch = nchunks - base_ch * NW
    max_ch = base_ch + 1
    mesh = plsc.VectorSubcoreMesh(core_axis_name="c", subcore_axis_name="s")
    out_type = [jax.ShapeDtypeStruct((NC, N_PAD, LANES), jnp.float32)]
    scratch = (
        [pltpu.VMEM_SHARED((N_PAD, LANES), jnp.float32)]
        + [pltpu.VMEM((chunk, LANES), jnp.float32)]
        + [pltpu.VMEM((chunk,), jnp.int32) for _ in range(nidx)]
        + [pltpu.SemaphoreType.DMA for _ in range(nsem + nidx)]
    )

    def body(ei, deg_out, dacc, ones, *refs):
        idx = refs[:nidx]
        sem_s = refs[nidx:nidx + nsem]
        sem_i = refs[nidx + nsem:]
        c = lax.axis_index("c")
        s = lax.axis_index("s")
        wid = c * NS + s
        nch = (base_ch + jnp.where(wid < extra_ch, 1, 0)
               if extra_ch else base_ch)
        start_w = wid * base_ch + jnp.minimum(wid, extra_ch)

        _zero_fill(ones, 128, LANES)
        r0 = s * ROWS_PER_TILE
        _zero_stripe(ones, 128, dacc, r0)

        one = jnp.zeros((LANES,), jnp.float32) + 1.0

        @pl.loop(0, chunk)
        def _(i):
            ones[i, pl.ds(0, LANES)] = one

        plsc.subcore_barrier()

        def issue_idx(k, ib):
            pltpu.async_copy(ei.at[2 * (start_w + k) + 1], idx[ib],
                             sem_i[ib])

        def wait_idx(k, ib):
            pltpu.make_async_copy(ei.at[2 * (start_w + k) + 1], idx[ib],
                                  sem_i[ib]).wait()

        def issue_scatter(sb, ib):
            pltpu.async_copy(ones, dacc.at[idx[ib]], sem_s[sb], add=True)

        def wait_scatter(sb, ib):
            pltpu.make_async_copy(ones, dacc.at[idx[ib]], sem_s[sb]).wait()

        issue_idx(0, 0)
        issue_idx(1, 1)

        nouter = (max_ch + nsem + nidx - 1) // nidx

        @pl.loop(0, nouter)
        def _(io):
            k0 = io * nidx
            for j in range(nidx):
                k = k0 + j
                sb = j % nsem

                @pl.when((k >= nsem) & (k - nsem < nch))
                def _():
                    wait_scatter(sb, (j - nsem) % nidx)

                @pl.when(k < nch)
                def _():
                    wait_idx(k, j)
                    issue_scatter(sb, j)

                @pl.when(k + 2 < nch)
                def _():
                    issue_idx(k + 2, (j + 2) % nidx)

        plsc.subcore_barrier()
        pltpu.sync_copy(dacc.at[pl.ds(r0, ROWS_PER_TILE)],
                        deg_out.at[c, pl.ds(r0, ROWS_PER_TILE)])

    return pl.kernel(
        body, out_type=out_type, mesh=mesh, scratch_types=scratch,
        compiler_params=pltpu.CompilerParams(use_tc_tiling_on_sc=False))


_sc_agg1 = _make_sc_agg(H, 2, 128)
_sc_agg2 = _make_sc_agg(D2, 2, 512)
_sc_deg = _make_sc_deg()

TC_BLK = 2000


def _lin2_body(x_ref, wl_ref, wr_ref, b_ref, xl_ref, xr_ref):
    x = x_ref[...]
    xl_ref[...] = jnp.dot(x, wl_ref[...], preferred_element_type=jnp.float32)
    xr_ref[...] = (jnp.dot(x, wr_ref[...], preferred_element_type=jnp.float32)
                   + b_ref[...])


def _lin2(x, wl, wr, b):
    return pl.pallas_call(
        _lin2_body,
        grid=(N // TC_BLK,),
        in_specs=[
            pl.BlockSpec((TC_BLK, F_IN), lambda i: (i, 0)),
            pl.BlockSpec((F_IN, H), lambda i: (0, 0)),
            pl.BlockSpec((F_IN, H), lambda i: (0, 0)),
            pl.BlockSpec((1, H), lambda i: (0, 0)),
        ],
        out_specs=[
            pl.BlockSpec((TC_BLK, H), lambda i: (i, 0)),
            pl.BlockSpec((TC_BLK, H), lambda i: (i, 0)),
        ],
        out_shape=[
            jax.ShapeDtypeStruct((N, H), jnp.float32),
            jax.ShapeDtypeStruct((N, H), jnp.float32),
        ],
    )(x, wl, wr, b)


def _mid_body(aggp_ref, degp_ref, xr_ref, w2l_ref, w2r_ref, b2_ref,
              hl_ref, hr_ref):
    agg = aggp_ref[0] + aggp_ref[1]
    deg = degp_ref[0, :, 0:1] + degp_ref[1, :, 0:1]
    h = jnp.maximum(agg / jnp.maximum(deg, 1.0) + xr_ref[...], 0.0)
    zcol = jnp.zeros((H, 1), jnp.float32)
    w2l = jnp.concatenate([w2l_ref[...], zcol], axis=1)
    w2r = jnp.concatenate([w2r_ref[...], zcol], axis=1)
    b2 = jnp.concatenate([b2_ref[...], jnp.zeros((1, 1), jnp.float32)], axis=1)
    hl_ref[...] = jnp.dot(h, w2l, preferred_element_type=jnp.float32)
    hr_ref[...] = jnp.dot(h, w2r, preferred_element_type=jnp.float32) + b2


def _mid(aggp, degp, xr, w2l, w2r, b2):
    return pl.pallas_call(
        _mid_body,
        grid=(N // TC_BLK,),
        in_specs=[
            pl.BlockSpec((NC, TC_BLK, H), lambda i: (0, i, 0)),
            pl.BlockSpec((NC, TC_BLK, LANES), lambda i: (0, i, 0)),
            pl.BlockSpec((TC_BLK, H), lambda i: (i, 0)),
            pl.BlockSpec((H, C_OUT), lambda i: (0, 0)),
            pl.BlockSpec((H, C_OUT), lambda i: (0, 0)),
            pl.BlockSpec((1, C_OUT), lambda i: (0, 0)),
        ],
        out_specs=[
            pl.BlockSpec((TC_BLK, D2), lambda i: (i, 0)),
            pl.BlockSpec((TC_BLK, D2), lambda i: (i, 0)),
        ],
        out_shape=[
            jax.ShapeDtypeStruct((N, D2), jnp.float32),
            jax.ShapeDtypeStruct((N, D2), jnp.float32),
        ],
    )(aggp, degp, xr, w2l, w2r, b2)


def _out_body(aggp_ref, degp_ref, hr_ref, o_ref):
    agg = aggp_ref[0] + aggp_ref[1]
    deg = degp_ref[0, :, 0:1] + degp_ref[1, :, 0:1]
    res = agg / jnp.maximum(deg, 1.0) + hr_ref[...]
    o_ref[...] = res[:, :C_OUT]


def _outk(agg2p, degp, hr):
    return pl.pallas_call(
        _out_body,
        grid=(N // TC_BLK,),
        in_specs=[
            pl.BlockSpec((NC, TC_BLK, D2), lambda i: (0, i, 0)),
            pl.BlockSpec((NC, TC_BLK, LANES), lambda i: (0, i, 0)),
            pl.BlockSpec((TC_BLK, D2), lambda i: (i, 0)),
        ],
        out_specs=pl.BlockSpec((TC_BLK, C_OUT), lambda i: (i, 0)),
        out_shape=jax.ShapeDtypeStruct((N, C_OUT), jnp.float32),
    )(agg2p, degp, hr)


@jax.jit
def kernel(x, edge_index, W1l, W1r, b1, W2l, W2r, b2):
    ei128 = _pack_ei(edge_index, 128)
    ei512 = _pack_ei(edge_index, 512)
    degp, = _sc_deg(ei512)
    xl, xr = _lin2(x, W1l, W1r, b1.reshape(1, H))
    agg1p, = _sc_agg1(xl, ei128)
    hl, hr = _mid(agg1p, degp, xr, W2l, W2r, b2.reshape(1, C_OUT))
    agg2p, = _sc_agg2(hl, ei512)
    return _outk(agg2p, degp, hr)


# speedup vs baseline: 1.0321x; 1.0002x over previous
import jax
import jax.numpy as jnp
from jax import lax
from jax.experimental import pallas as pl
from jax.experimental.pallas import tpu as pltpu
from jax.experimental.pallas import tpu_sc as plsc

N = 10000
E = 320000
F_IN = 128
H = 128
C_OUT = 47
D2 = 48

NC = 2
NS = 16
NW = NC * NS
LANES = 16

N_PAD = 10240
ROWS_PER_TILE = N_PAD // NS

def _zero_fill(buf, rows, d):
    zero = jnp.zeros((LANES,), jnp.float32)

    @pl.loop(0, rows)
    def _(i):
        @pl.loop(0, d // LANES)
        def _(j):
            buf[i, pl.ds(j * LANES, LANES)] = zero


def _zero_stripe(buf, rows, acc, r0):
    @pl.loop(0, ROWS_PER_TILE // rows)
    def _(k):
        pltpu.sync_copy(buf.at[pl.ds(0, rows)],
                        acc.at[pl.ds(r0 + k * rows, rows)])


def _pack_ei(edge_index, chunk):
    nck = E // chunk
    e3 = edge_index.reshape(2, nck, chunk).transpose(1, 0, 2)
    return jnp.pad(e3.reshape(2 * nck, chunk), ((0, 8), (0, 0)))


UNROLL = 16
NBAT = 4


def _make_sc_agg(d, nbuf, chunk=128):
    assert UNROLL % nbuf == 0 and d % LANES == 0
    nchunks = E // chunk
    base_ch = nchunks // NW
    extra_ch = nchunks - base_ch * NW
    max_ch = base_ch + 1

    mesh = plsc.VectorSubcoreMesh(core_axis_name="c", subcore_axis_name="s")
    out_type = [jax.ShapeDtypeStruct((NC, N_PAD, d), jnp.float32)]
    scratch = (
        [pltpu.VMEM_SHARED((N_PAD, d), jnp.float32)]
        + [pltpu.VMEM((chunk, d), jnp.float32) for _ in range(nbuf)]
        + [pltpu.VMEM((8, chunk), jnp.int32) for _ in range(NBAT)]
        + [pltpu.SemaphoreType.DMA for _ in range(2 * nbuf + NBAT)]
    )

    def body(xw, ei, agg_out, acc, *refs):
        rows = refs[:nbuf]
        idxb = refs[nbuf:nbuf + NBAT]
        sem_g = refs[nbuf + NBAT:2 * nbuf + NBAT]
        sem_s = refs[2 * nbuf + NBAT:3 * nbuf + NBAT]
        sem_i = refs[3 * nbuf + NBAT:]
        c = lax.axis_index("c")
        s = lax.axis_index("s")
        wid = c * NS + s
        nch = (base_ch + jnp.where(wid < extra_ch, 1, 0)
               if extra_ch else base_ch)
        start_w = wid * base_ch + jnp.minimum(wid, extra_ch)

        _zero_fill(rows[0], 128, d)
        r0 = s * ROWS_PER_TILE
        _zero_stripe(rows[0], 128, acc, r0)
        plsc.subcore_barrier()

        def issue_batch(k_first, ring):
            base = 2 * (start_w + k_first)
            pltpu.async_copy(ei.at[pl.ds(base, 8)], idxb[ring], sem_i[ring])

        def wait_batch(k_first, ring):
            base = 2 * (start_w + k_first)
            pltpu.make_async_copy(ei.at[pl.ds(base, 8)], idxb[ring],
                                  sem_i[ring]).wait()

        def _sidx(ring, t):
            return idxb[ring].at[2 * t]

        def _didx(ring, t):
            return idxb[ring].at[2 * t + 1]

        def issue_gather(b, ring, t):
            pltpu.async_copy(xw.at[_sidx(ring, t)], rows[b], sem_g[b])

        def wait_gather(b, ring, t):
            pltpu.make_async_copy(xw.at[_sidx(ring, t)], rows[b],
                                  sem_g[b]).wait()

        def issue_scatter(b, ring, t):
            pltpu.async_copy(rows[b], acc.at[_didx(ring, t)],
                             sem_s[b], add=True)

        def wait_scatter(b, ring, t):
            pltpu.make_async_copy(rows[b], acc.at[_didx(ring, t)],
                                  sem_s[b]).wait()

        issue_batch(0, 0)
        issue_batch(4, 1)
        wait_batch(0, 0)
        issue_gather(0, 0, 0)

        nouter = (max_ch + nbuf + UNROLL - 1) // UNROLL

        @pl.loop(0, nouter)
        def _(io):
            k0 = io * UNROLL
            for j in range(UNROLL):
                k = k0 + j
                b = j % nbuf
                ring = (j // 4) % NBAT
                t = j % 4
                jn = (j + 1) % UNROLL
                jp = (j + 1 - nbuf) % UNROLL

                @pl.when(k < nch)
                def _():
                    wait_gather(b, ring, t)
                    issue_scatter(b, ring, t)

                @pl.when((k + 1 - nbuf >= 0) & (k + 1 - nbuf < nch))
                def _():
                    wait_scatter((j + 1) % nbuf, (jp // 4) % NBAT, jp % 4)

                if (j + 1) % 4 == 0:
                    @pl.when(k + 1 < nch)
                    def _():
                        wait_batch(k + 1, (jn // 4) % NBAT)

                    @pl.when(k + 5 < nch)
                    def _():
                        issue_batch(k + 5, ((j + 1) // 4 + 1) % NBAT)

                @pl.when(k + 1 < nch)
                def _():
                    issue_gather((j + 1) % nbuf, (jn // 4) % NBAT, jn % 4)

        plsc.subcore_barrier()

        pltpu.sync_copy(acc.at[pl.ds(r0, ROWS_PER_TILE)],
                        agg_out.at[c, pl.ds(r0, ROWS_PER_TILE)])

    return pl.kernel(
        body, out_type=out_type, mesh=mesh, scratch_types=scratch,
        compiler_params=pltpu.CompilerParams(use_tc_tiling_on_sc=False))


def _make_sc_deg(chunk=512, nsem=4, nidx=8):
    nchunks = E // chunk
    base_ch = nchunks // NW
    extra_ch = nchunks - base_ch * NW
    max_ch = base_ch + 1
    mesh = plsc.VectorSubcoreMesh(core_axis_name="c", subcore_axis_name="s")
    out_type = [jax.ShapeDtypeStruct((NC, N_PAD, LANES), jnp.float32)]
    scratch = (
        [pltpu.VMEM_SHARED((N_PAD, LANES), jnp.float32)]
        + [pltpu.VMEM((chunk, LANES), jnp.float32)]
        + [pltpu.VMEM((chunk,), jnp.int32) for _ in range(nidx)]
        + [pltpu.SemaphoreType.DMA for _ in range(nsem + nidx)]
    )

    def body(ei, deg_out, dacc, ones, *refs):
        idx = refs[:nidx]
        sem_s = refs[nidx:nidx + nsem]
        sem_i = refs[nidx + nsem:]
        c = lax.axis_index("c")
        s = lax.axis_index("s")
        wid = c * NS + s
        nch = (base_ch + jnp.where(wid < extra_ch, 1, 0)
               if extra_ch else base_ch)
        start_w = wid * base_ch + jnp.minimum(wid, extra_ch)

        _zero_fill(ones, 128, LANES)
        r0 = s * ROWS_PER_TILE
        _zero_stripe(ones, 128, dacc, r0)

        one = jnp.zeros((LANES,), jnp.float32) + 1.0

        @pl.loop(0, chunk)
        def _(i):
            ones[i, pl.ds(0, LANES)] = one

        plsc.subcore_barrier()

        def issue_idx(k, ib):
            pltpu.async_copy(ei.at[2 * (start_w + k) + 1], idx[ib],
                             sem_i[ib])

        def wait_idx(k, ib):
            pltpu.make_async_copy(ei.at[2 * (start_w + k) + 1], idx[ib],
                                  sem_i[ib]).wait()

        def issue_scatter(sb, ib):
            pltpu.async_copy(ones, dacc.at[idx[ib]], sem_s[sb], add=True)

        def wait_scatter(sb, ib):
            pltpu.make_async_copy(ones, dacc.at[idx[ib]], sem_s[sb]).wait()

        issue_idx(0, 0)
        issue_idx(1, 1)

        nouter = (max_ch + nsem + nidx - 1) // nidx

        @pl.loop(0, nouter)
        def _(io):
            k0 = io * nidx
            for j in range(nidx):
                k = k0 + j
                sb = j % nsem

                @pl.when((k >= nsem) & (k - nsem < nch))
                def _():
                    wait_scatter(sb, (j - nsem) % nidx)

                @pl.when(k < nch)
                def _():
                    wait_idx(k, j)
                    issue_scatter(sb, j)

                @pl.when(k + 2 < nch)
                def _():
                    issue_idx(k + 2, (j + 2) % nidx)

        plsc.subcore_barrier()
        pltpu.sync_copy(dacc.at[pl.ds(r0, ROWS_PER_TILE)],
                        deg_out.at[c, pl.ds(r0, ROWS_PER_TILE)])

    return pl.kernel(
        body, out_type=out_type, mesh=mesh, scratch_types=scratch,
        compiler_params=pltpu.CompilerParams(use_tc_tiling_on_sc=False))


_sc_agg1 = _make_sc_agg(H, 2, 128)
_sc_agg2 = _make_sc_agg(D2, 2, 512)
_sc_deg = _make_sc_deg()

TC_BLK = 2000


def _lin2_body(x_ref, wl_ref, wr_ref, b_ref, xl_ref, xr_ref):
    x = x_ref[...]
    xl_ref[...] = jnp.dot(x, wl_ref[...], preferred_element_type=jnp.float32)
    xr_ref[...] = (jnp.dot(x, wr_ref[...], preferred_element_type=jnp.float32)
                   + b_ref[...])


def _lin2(x, wl, wr, b):
    return pl.pallas_call(
        _lin2_body,
        grid=(N // TC_BLK,),
        in_specs=[
            pl.BlockSpec((TC_BLK, F_IN), lambda i: (i, 0)),
            pl.BlockSpec((F_IN, H), lambda i: (0, 0)),
            pl.BlockSpec((F_IN, H), lambda i: (0, 0)),
            pl.BlockSpec((1, H), lambda i: (0, 0)),
        ],
        out_specs=[
            pl.BlockSpec((TC_BLK, H), lambda i: (i, 0)),
            pl.BlockSpec((TC_BLK, H), lambda i: (i, 0)),
        ],
        out_shape=[
            jax.ShapeDtypeStruct((N, H), jnp.float32),
            jax.ShapeDtypeStruct((N, H), jnp.float32),
        ],
    )(x, wl, wr, b)


def _mid_body(aggp_ref, degp_ref, xr_ref, w2l_ref, w2r_ref, b2_ref,
              hl_ref, hr_ref):
    agg = aggp_ref[0] + aggp_ref[1]
    deg = degp_ref[0, :, 0:1] + degp_ref[1, :, 0:1]
    h = jnp.maximum(agg / jnp.maximum(deg, 1.0) + xr_ref[...], 0.0)
    zcol = jnp.zeros((H, 1), jnp.float32)
    w2l = jnp.concatenate([w2l_ref[...], zcol], axis=1)
    w2r = jnp.concatenate([w2r_ref[...], zcol], axis=1)
    b2 = jnp.concatenate([b2_ref[...], jnp.zeros((1, 1), jnp.float32)], axis=1)
    hl_ref[...] = jnp.dot(h, w2l, preferred_element_type=jnp.float32)
    hr_ref[...] = jnp.dot(h, w2r, preferred_element_type=jnp.float32) + b2


def _mid(aggp, degp, xr, w2l, w2r, b2):
    return pl.pallas_call(
        _mid_body,
        grid=(N // TC_BLK,),
        in_specs=[
            pl.BlockSpec((NC, TC_BLK, H), lambda i: (0, i, 0)),
            pl.BlockSpec((NC, TC_BLK, LANES), lambda i: (0, i, 0)),
            pl.BlockSpec((TC_BLK, H), lambda i: (i, 0)),
            pl.BlockSpec((H, C_OUT), lambda i: (0, 0)),
            pl.BlockSpec((H, C_OUT), lambda i: (0, 0)),
            pl.BlockSpec((1, C_OUT), lambda i: (0, 0)),
        ],
        out_specs=[
            pl.BlockSpec((TC_BLK, D2), lambda i: (i, 0)),
            pl.BlockSpec((TC_BLK, D2), lambda i: (i, 0)),
        ],
        out_shape=[
            jax.ShapeDtypeStruct((N, D2), jnp.float32),
            jax.ShapeDtypeStruct((N, D2), jnp.float32),
        ],
    )(aggp, degp, xr, w2l, w2r, b2)


def _out_body(aggp_ref, degp_ref, hr_ref, o_ref):
    agg = aggp_ref[0] + aggp_ref[1]
    deg = degp_ref[0, :, 0:1] + degp_ref[1, :, 0:1]
    res = agg / jnp.maximum(deg, 1.0) + hr_ref[...]
    o_ref[...] = res[:, :C_OUT]


def _outk(agg2p, degp, hr):
    return pl.pallas_call(
        _out_body,
        grid=(N // TC_BLK,),
        in_specs=[
            pl.BlockSpec((NC, TC_BLK, D2), lambda i: (0, i, 0)),
            pl.BlockSpec((NC, TC_BLK, LANES), lambda i: (0, i, 0)),
            pl.BlockSpec((TC_BLK, D2), lambda i: (i, 0)),
        ],
        out_specs=pl.BlockSpec((TC_BLK, C_OUT), lambda i: (i, 0)),
        out_shape=jax.ShapeDtypeStruct((N, C_OUT), jnp.float32),
    )(agg2p, degp, hr)


@jax.jit
def kernel(x, edge_index, W1l, W1r, b1, W2l, W2r, b2):
    ei128 = _pack_ei(edge_index, 128)
    ei512 = _pack_ei(edge_index, 512)
    degp, = _sc_deg(ei512)
    xl, xr = _lin2(x, W1l, W1r, b1.reshape(1, H))
    agg1p, = _sc_agg1(xl, ei128)
    hl, hr = _mid(agg1p, degp, xr, W2l, W2r, b2.reshape(1, C_OUT))
    agg2p, = _sc_agg2(hl, ei512)
    return _outk(agg2p, degp, hr)
